# baseline (speedup 1.0000x reference)
import jax
import jax.numpy as jnp
from jax import lax
from jax.experimental import pallas as pl
from jax.experimental.pallas import tpu as pltpu
from jax.experimental.pallas import tpu_sc as plsc

_NC = 2
_NS = 16
_NW = _NC * _NS
_LANES = 16

_B = 4096
_E = 64
_H = 50
_V = 1000000
_B_PER_W = _B // _NW
_CHUNKS = _B_PER_W
_NBUF = 4
_QS = _E // _LANES

_DB = 256
_NDB = 3906
_DTAIL = _V - _NDB * _DB
_DGR = _DB // _LANES
_KMAX = 124


def _detile_body(tt_hbm, out_hbm, buf_a, buf_b, buf1_a, buf1_b, tb_v, tb1_v,
                 si0, si1, so0, so1):
    sin = (si0, si1)
    sout = (so0, so1)
    bufs = (buf_a, buf_b)
    buf1s = (buf1_a, buf1_b)
    wid = lax.axis_index("s") * _NC + lax.axis_index("c")
    lane64 = lax.iota(jnp.int32, _LANES) * _E

    def blk_of(k):
        return jnp.minimum(wid + _NW * k, _NDB - 1)

    def valid(k):
        return (wid + _NW * k) < _NDB

    def start_in(k, bb):
        t0 = pl.multiple_of(blk_of(k) * _DB, 8)
        pltpu.async_copy(tt_hbm.at[:, pl.ds(t0, _DB)], bufs[bb], sin[bb])

    def wait_in(bb):
        pltpu.make_async_copy(
            tt_hbm.at[:, pl.ds(0, _DB)], bufs[bb], sin[bb]
        ).wait()

    def start_out(k, bb):
        o0 = pl.multiple_of(blk_of(k) * (_DB * _E), 8)
        pltpu.async_copy(buf1s[bb], out_hbm.at[pl.ds(o0, _DB * _E)],
                         sout[bb])

    def wait_out(bb):
        pltpu.make_async_copy(
            buf1s[bb], out_hbm.at[pl.ds(0, _DB * _E)], sout[bb]
        ).wait()

    def transpose_block(src, dst):
        def tg_body(tg, c):
            base = tg * (_LANES * _E) + lane64
            for d in range(_E):
                val = src[d, pl.ds(tg * _LANES, _LANES)]
                plsc.store_scatter(dst, [base + d], val)
            return c

        plsc.parallel_loop(0, src.shape[1] // _LANES, carry=jnp.int32(0))(
            tg_body)

    for bb in range(2):
        start_in(bb, bb)

    def k_body(k2, carry):
        for bb in range(2):
            k = k2 * 2 + bb
            wait_in(bb)

            @pl.when(k >= 2)
            def _():
                wait_out(bb)

            transpose_block(bufs[bb], buf1s[bb])

            @pl.when(valid(k))
            def _():
                start_out(k, bb)

            @pl.when(k + 2 < _KMAX)
            def _():
                start_in(k + 2, bb)

        return carry

    lax.fori_loop(0, _KMAX // 2, k_body, 0)
    for bb in range(2):
        @pl.when(valid(_KMAX - 2 + bb))
        def _():
            wait_out(bb)

    @pl.when(wid == 2)
    def _():
        t0 = _NDB * _DB
        pltpu.sync_copy(tt_hbm.at[:, pl.ds(t0, _DTAIL)], tb_v)
        transpose_block(tb_v, tb1_v)
        pltpu.sync_copy(tb1_v, out_hbm.at[pl.ds(t0 * _E, _DTAIL * _E)])


def _detile(tt):
    k = pl.kernel(
        _detile_body,
        out_type=jax.ShapeDtypeStruct((_V * _E,), jnp.float32),
        mesh=plsc.VectorSubcoreMesh(
            core_axis_name="c", subcore_axis_name="s",
            num_cores=_NC, num_subcores=_NS,
        ),
        scratch_types=[
            pltpu.VMEM((_E, _DB), jnp.float32),
            pltpu.VMEM((_E, _DB), jnp.float32),
            pltpu.VMEM((_DB * _E,), jnp.float32),
            pltpu.VMEM((_DB * _E,), jnp.float32),
            pltpu.VMEM((_E, _DTAIL), jnp.float32),
            pltpu.VMEM((_DTAIL * _E,), jnp.float32),
            pltpu.SemaphoreType.DMA,
            pltpu.SemaphoreType.DMA,
            pltpu.SemaphoreType.DMA,
            pltpu.SemaphoreType.DMA,
        ],
        compiler_params=pltpu.CompilerParams(
            use_tc_tiling_on_sc=True, needs_layout_passes=False,
        ),
    )
    return k(tt)


def _gather_sum_body(idx_hbm, table_hbm, out_hbm,
                     idx_v, rows_v, outb_v, s0, s1, s2, s3):
    sems = (s0, s1, s2, s3)
    wid = lax.axis_index("s") * _NC + lax.axis_index("c")
    base = pl.multiple_of(wid * _B_PER_W, 8)

    pltpu.sync_copy(idx_hbm.at[pl.ds(base, _B_PER_W)], idx_v)

    def gather_start(c, b):
        pltpu.async_copy(table_hbm.at[idx_v.at[c]], rows_v.at[b], sems[b])

    def gather_wait(c, b):
        pltpu.make_async_copy(
            table_hbm.at[idx_v.at[c]], rows_v.at[b], sems[b]
        ).wait()

    for b in range(_NBUF):
        gather_start(b, b)

    def reduce_rows(rb):
        zero = jnp.zeros((_LANES,), jnp.float32)
        init = (zero, zero, zero, zero)

        def red(j, acc):
            return tuple(
                acc[q] + rb[j, pl.ds(q * _LANES, _LANES)] for q in range(_QS)
            )

        return plsc.parallel_loop(0, _H, unroll=10, carry=init)(red)

    def t_body(t, carry):
        for b in range(_NBUF):
            c = t * _NBUF + b
            gather_wait(c, b)
            acc = reduce_rows(rows_v.at[b])
            for q in range(_QS):
                outb_v[c, pl.ds(q * _LANES, _LANES)] = acc[q]
            nc = c + _NBUF

            @pl.when(nc < _CHUNKS)
            def _():
                gather_start(nc, b)

        return carry

    lax.fori_loop(0, _CHUNKS // _NBUF, t_body, 0)

    pltpu.sync_copy(outb_v, out_hbm.at[pl.ds(base, _B_PER_W)])


def _gather_sum(idx, table2):
    k = pl.kernel(
        _gather_sum_body,
        out_type=jax.ShapeDtypeStruct((_B, _E), jnp.float32),
        mesh=plsc.VectorSubcoreMesh(
            core_axis_name="c", subcore_axis_name="s",
            num_cores=_NC, num_subcores=_NS,
        ),
        scratch_types=[
            pltpu.VMEM((_B_PER_W, _H), jnp.int32),
            pltpu.VMEM((_NBUF, _H, _E), jnp.float32),
            pltpu.VMEM((_B_PER_W, _E), jnp.float32),
            pltpu.SemaphoreType.DMA,
            pltpu.SemaphoreType.DMA,
            pltpu.SemaphoreType.DMA,
            pltpu.SemaphoreType.DMA,
        ],
        compiler_params=pltpu.CompilerParams(use_tc_tiling_on_sc=False),
    )
    return k(idx, table2)


def _proj_body(x_ref, w_ref, b_ref, o_ref):
    x = jnp.maximum(x_ref[...], 0.0)
    o_ref[...] = (
        lax.dot_general(
            w_ref[...], x,
            dimension_numbers=(((1,), (1,)), ((), ())),
            preferred_element_type=jnp.float32,
        )
        + b_ref[...]
    )


def _proj_t(x, W, bcol):
    B, E = x.shape
    T = W.shape[0]
    blk = 512
    return pl.pallas_call(
        _proj_body,
        grid=(B // blk,),
        in_specs=[
            pl.BlockSpec((blk, E), lambda i: (i, 0)),
            pl.BlockSpec((T, E), lambda i: (0, 0)),
            pl.BlockSpec((T, 1), lambda i: (0, 0)),
        ],
        out_specs=pl.BlockSpec((T, blk), lambda i: (0, i)),
        out_shape=jax.ShapeDtypeStruct((T, B), jnp.float32),
    )(x, W, bcol)


def kernel(input_text, table, W, b):
    table2 = _detile(table.T).reshape(_V, _E)
    sums = _gather_sum(input_text, table2)
    out_t = _proj_t(sums, W, b.reshape(-1, 1))
    return out_t.T


# speedup vs baseline: 1.8716x; 1.8716x over previous
import jax
import jax.numpy as jnp
from jax import lax
from jax.experimental import pallas as pl
from jax.experimental.pallas import tpu as pltpu
from jax.experimental.pallas import tpu_sc as plsc

_NC = 2
_NS = 16
_NW = _NC * _NS
_LANES = 16

_B = 4096
_E = 64
_H = 50
_V = 1000000
_B_PER_W = _B // _NW
_CHUNKS = _B_PER_W
_NBUF = 4
_QS = _E // _LANES

def _gather_sum_body(idx_hbm, table_hbm, out_hbm,
                     raw_v, idx_v, rows_v, outb_v, s0, s1, s2, s3):
    sems = (s0, s1, s2, s3)
    wid = lax.axis_index("s") * _NC + lax.axis_index("c")
    base = pl.multiple_of(wid * _B_PER_W, 8)

    pltpu.sync_copy(idx_hbm.at[pl.ds(base, _B_PER_W)], raw_v)

    def conv(r, carry):
        for col in (0, 16, 32, _H - _LANES):
            v = raw_v[r, pl.ds(col, _LANES)]
            idx_v[r, pl.ds(col, _LANES)] = v + v
        return carry

    plsc.parallel_loop(0, _B_PER_W, unroll=4, carry=jnp.int32(0))(conv)

    def gather_start(c, b):
        pltpu.async_copy(table_hbm.at[idx_v.at[c]], rows_v.at[b], sems[b])

    def gather_wait(c, b):
        pltpu.make_async_copy(
            table_hbm.at[idx_v.at[c]], rows_v.at[b], sems[b]
        ).wait()

    for b in range(_NBUF):
        gather_start(b, b)

    def reduce_rows(rb):
        zero = jnp.zeros((_LANES,), jnp.float32)
        init = (zero, zero, zero, zero)

        def red(j, acc):
            return tuple(
                acc[q] + rb[j, pl.ds(q * _LANES, _LANES)] for q in range(_QS)
            )

        return plsc.parallel_loop(0, _H, unroll=10, carry=init)(red)

    def t_body(t, carry):
        for b in range(_NBUF):
            c = t * _NBUF + b
            gather_wait(c, b)
            acc = reduce_rows(rows_v.at[b])
            for q in range(_QS):
                outb_v[c, pl.ds(q * _LANES, _LANES)] = acc[q]
            nc = c + _NBUF

            @pl.when(nc < _CHUNKS)
            def _():
                gather_start(nc, b)

        return carry

    lax.fori_loop(0, _CHUNKS // _NBUF, t_body, 0)

    pltpu.sync_copy(outb_v, out_hbm.at[pl.ds(base, _B_PER_W)])


def _gather_sum(idx, table2):
    k = pl.kernel(
        _gather_sum_body,
        out_type=jax.ShapeDtypeStruct((_B, _E), jnp.float32),
        mesh=plsc.VectorSubcoreMesh(
            core_axis_name="c", subcore_axis_name="s",
            num_cores=_NC, num_subcores=_NS,
        ),
        scratch_types=[
            pltpu.VMEM((_B_PER_W, _H), jnp.int32),
            pltpu.VMEM((_B_PER_W, _H), jnp.int32),
            pltpu.VMEM((_NBUF, _H, _E), jnp.float32),
            pltpu.VMEM((_B_PER_W, _E), jnp.float32),
            pltpu.SemaphoreType.DMA,
            pltpu.SemaphoreType.DMA,
            pltpu.SemaphoreType.DMA,
            pltpu.SemaphoreType.DMA,
        ],
        compiler_params=pltpu.CompilerParams(use_tc_tiling_on_sc=False),
    )
    return k(idx, table2)


def _proj_body(x_ref, w_ref, b_ref, o_ref):
    x = jnp.maximum(x_ref[...], 0.0)
    o_ref[...] = (
        lax.dot_general(
            w_ref[...], x,
            dimension_numbers=(((1,), (1,)), ((), ())),
            preferred_element_type=jnp.float32,
        )
        + b_ref[...]
    )


def _proj_t(x, W, bcol):
    B, E = x.shape
    T = W.shape[0]
    blk = 512
    return pl.pallas_call(
        _proj_body,
        grid=(B // blk,),
        in_specs=[
            pl.BlockSpec((blk, E), lambda i: (i, 0)),
            pl.BlockSpec((T, E), lambda i: (0, 0)),
            pl.BlockSpec((T, 1), lambda i: (0, 0)),
        ],
        out_specs=pl.BlockSpec((T, blk), lambda i: (0, i)),
        out_shape=jax.ShapeDtypeStruct((T, B), jnp.float32),
    )(x, W, bcol)


def kernel(input_text, table, W, b):
    table2 = jnp.pad(table, ((0, 0), (0, _E))).reshape(2 * _V, _E)
    sums = _gather_sum(input_text, table2)
    out_t = _proj_t(sums, W, b.reshape(-1, 1))
    return out_t.T


# speedup vs baseline: 1.8886x; 1.0091x over previous
import jax
import jax.numpy as jnp
from jax import lax
from jax.experimental import pallas as pl
from jax.experimental.pallas import tpu as pltpu
from jax.experimental.pallas import tpu_sc as plsc

_NC = 2
_NS = 16
_NW = _NC * _NS
_LANES = 16

_B = 4096
_E = 64
_H = 50
_V = 1000000
_B_PER_W = _B // _NW
_CH = 2
_CHUNKS = _B_PER_W // _CH
_NBUF = 4
_QS = _E // _LANES

def _gather_sum_body(idx_hbm, table_hbm, out_hbm,
                     raw_v, idx_v, rows_v, outb_v, s0, s1, s2, s3):
    sems = (s0, s1, s2, s3)
    wid = lax.axis_index("s") * _NC + lax.axis_index("c")
    base = pl.multiple_of(wid * _B_PER_W, 8)

    pltpu.sync_copy(idx_hbm.at[pl.ds(base, _B_PER_W)], raw_v)

    def conv(e, carry):
        c = lax.shift_right_logical(e, 1)
        pbase = lax.bitwise_and(e, jnp.int32(1)) * _H
        for col in (0, 16, 32, _H - _LANES):
            v = raw_v[e, pl.ds(col, _LANES)]
            idx_v[c, pl.ds(pbase + col, _LANES)] = v + v
        return carry

    plsc.parallel_loop(0, _B_PER_W, unroll=4, carry=jnp.int32(0))(conv)

    def gather_start(c, b):
        pltpu.async_copy(table_hbm.at[idx_v.at[c]], rows_v.at[b], sems[b])

    def gather_wait(c, b):
        pltpu.make_async_copy(
            table_hbm.at[idx_v.at[c]], rows_v.at[b], sems[b]
        ).wait()

    for b in range(_NBUF):
        gather_start(b, b)

    def reduce_rows(rb, r0):
        zero = jnp.zeros((_LANES,), jnp.float32)
        init = (zero, zero, zero, zero)

        def red(j, acc):
            return tuple(
                acc[q] + rb[r0 + j, pl.ds(q * _LANES, _LANES)]
                for q in range(_QS)
            )

        return plsc.parallel_loop(0, _H, unroll=10, carry=init)(red)

    def t_body(t, carry):
        for b in range(_NBUF):
            c = t * _NBUF + b
            gather_wait(c, b)
            for e in range(_CH):
                acc = reduce_rows(rows_v.at[b], e * _H)
                row = c * _CH + e
                for q in range(_QS):
                    outb_v[row, pl.ds(q * _LANES, _LANES)] = acc[q]
            nc = c + _NBUF

            @pl.when(nc < _CHUNKS)
            def _():
                gather_start(nc, b)

        return carry

    lax.fori_loop(0, _CHUNKS // _NBUF, t_body, 0)

    pltpu.sync_copy(outb_v, out_hbm.at[pl.ds(base, _B_PER_W)])


def _gather_sum(idx, table2):
    k = pl.kernel(
        _gather_sum_body,
        out_type=jax.ShapeDtypeStruct((_B, _E), jnp.float32),
        mesh=plsc.VectorSubcoreMesh(
            core_axis_name="c", subcore_axis_name="s",
            num_cores=_NC, num_subcores=_NS,
        ),
        scratch_types=[
            pltpu.VMEM((_B_PER_W, _H), jnp.int32),
            pltpu.VMEM((_CHUNKS, _CH * _H), jnp.int32),
            pltpu.VMEM((_NBUF, _CH * _H, _E), jnp.float32),
            pltpu.VMEM((_B_PER_W, _E), jnp.float32),
            pltpu.SemaphoreType.DMA,
            pltpu.SemaphoreType.DMA,
            pltpu.SemaphoreType.DMA,
            pltpu.SemaphoreType.DMA,
        ],
        compiler_params=pltpu.CompilerParams(use_tc_tiling_on_sc=False),
    )
    return k(idx, table2)


def _proj_body(x_ref, w_ref, b_ref, o_ref):
    x = jnp.maximum(x_ref[...], 0.0)
    o_ref[...] = (
        lax.dot_general(
            w_ref[...], x,
            dimension_numbers=(((1,), (1,)), ((), ())),
            preferred_element_type=jnp.float32,
        )
        + b_ref[...]
    )


def _proj_t(x, W, bcol):
    B, E = x.shape
    T = W.shape[0]
    blk = 512
    return pl.pallas_call(
        _proj_body,
        grid=(B // blk,),
        in_specs=[
            pl.BlockSpec((blk, E), lambda i: (i, 0)),
            pl.BlockSpec((T, E), lambda i: (0, 0)),
            pl.BlockSpec((T, 1), lambda i: (0, 0)),
        ],
        out_specs=pl.BlockSpec((T, blk), lambda i: (0, i)),
        out_shape=jax.ShapeDtypeStruct((T, B), jnp.float32),
    )(x, W, bcol)


def kernel(input_text, table, W, b):
    table2 = jnp.pad(table, ((0, 0), (0, _E))).reshape(2 * _V, _E)
    sums = _gather_sum(input_text, table2)
    out_t = _proj_t(sums, W, b.reshape(-1, 1))
    return out_t.T
